# Initial kernel scaffold; baseline (speedup 1.0000x reference)
import functools

import jax
import jax.numpy as jnp
from jax import lax
from jax.experimental import pallas as pl
from jax.experimental.pallas import tpu as pltpu
from jax.experimental.pallas import tpu_sc as plsc

N_NODES = 10000
N_EDGES = 320000
GROUP = 128
NC, NS = 2, 16
NW = NC * NS
GPW = -(-N_EDGES // (GROUP * NW))
E_PAD = GPW * NW * GROUP
ACC_ROWS = 10240
RPS = N_NODES // NS
ROW_BLK = 1000


def _sc_mesh():
    return plsc.VectorSubcoreMesh(core_axis_name="c", subcore_axis_name="s")


def _make_sc_degree():
    @functools.partial(
        pl.kernel,
        mesh=_sc_mesh(),
        out_type=jax.ShapeDtypeStruct((2 * N_NODES, 16), jnp.float32),
        scratch_types=[
            pltpu.VMEM((GPW, GROUP), jnp.int32),
            pltpu.VMEM((GROUP, 16), jnp.float32),
            pltpu.VMEM_SHARED((ACC_ROWS, 16), jnp.float32),
        ],
    )
    def k(dst_hbm, ones_hbm, zeros_hbm, out_hbm, didx, ones_v, acc):
        c = lax.axis_index("c")
        s = lax.axis_index("s")
        wid = s * NC + c
        pltpu.sync_copy(zeros_hbm, ones_v)
        for t in range(5):
            pltpu.sync_copy(ones_v, acc.at[pl.ds(s * 640 + t * GROUP, GROUP)])
        plsc.subcore_barrier()
        pltpu.sync_copy(ones_hbm, ones_v)
        pltpu.sync_copy(dst_hbm.at[pl.ds(wid * GPW, GPW)], didx)

        def body(t, carry):
            pltpu.sync_copy(ones_v, acc.at[didx.at[t]], add=True)
            return carry

        lax.fori_loop(0, GPW, body, 0)
        plsc.subcore_barrier()
        for t in range(5):
            start = s * RPS + t * 125
            pltpu.sync_copy(acc.at[pl.ds(start, 125)], ones_v.at[pl.ds(0, 125)])
            pltpu.sync_copy(ones_v.at[pl.ds(0, 125)],
                            out_hbm.at[pl.ds(c * N_NODES + start, 125)])

    return k


def _make_sc_prop(d):
    @functools.partial(
        pl.kernel,
        mesh=_sc_mesh(),
        out_type=jax.ShapeDtypeStruct((2 * N_NODES, d), jnp.float32),
        scratch_types=[
            pltpu.VMEM((GPW, GROUP), jnp.int32),
            pltpu.VMEM((GPW, GROUP), jnp.int32),
            pltpu.VMEM((GROUP, d), jnp.float32),
            pltpu.VMEM_SHARED((ACC_ROWS, d), jnp.float32),
            pltpu.SemaphoreType.DMA,
        ],
    )
    def k(g_hbm, src_hbm, dst_hbm, zeros_hbm, out_hbm,
          sidx, didx, rows_v, acc, sem):
        c = lax.axis_index("c")
        s = lax.axis_index("s")
        wid = s * NC + c
        pltpu.sync_copy(zeros_hbm, rows_v)
        for t in range(5):
            pltpu.sync_copy(rows_v, acc.at[pl.ds(s * 640 + t * GROUP, GROUP)])
        plsc.subcore_barrier()
        pltpu.sync_copy(src_hbm.at[pl.ds(wid * GPW, GPW)], sidx)
        pltpu.sync_copy(dst_hbm.at[pl.ds(wid * GPW, GPW)], didx)

        def body(t, carry):
            pltpu.async_copy(g_hbm.at[sidx.at[t]], rows_v, sem).wait()
            pltpu.sync_copy(rows_v, acc.at[didx.at[t]], add=True)
            return carry

        lax.fori_loop(0, GPW, body, 0)
        plsc.subcore_barrier()
        for t in range(5):
            start = s * RPS + t * 125
            pltpu.sync_copy(acc.at[pl.ds(start, 125)], rows_v.at[pl.ds(0, 125)])
            pltpu.sync_copy(rows_v.at[pl.ds(0, 125)],
                            out_hbm.at[pl.ds(c * N_NODES + start, 125)])

    return k


def _deg_dinv(d0, d1):
    deg = d0[:, 0:1] + d1[:, 0:1]
    return jnp.where(deg > 0, lax.rsqrt(jnp.maximum(deg, 1.0)), 0.0), deg


def _scale_x_body(d0, d1, x_ref, g_ref):
    dinv, _ = _deg_dinv(d0[...], d1[...])
    g_ref[...] = x_ref[...] * dinv


def _rescale_body(d0, d1, s0, s1, g_ref):
    _, deg = _deg_dinv(d0[...], d1[...])
    inv = jnp.where(deg > 0, 1.0 / jnp.maximum(deg, 1.0), 0.0)
    g_ref[...] = (s0[...] + s1[...]) * inv


def _tag_out_body(use_elu, d0, d1, h_ref, s1a, s1b, s2a, s2b, s3a, s3b,
                  w_ref, b_ref, out_ref, gnext_ref):
    dinv, _ = _deg_dinv(d0[...], d1[...])
    h1 = (s1a[...] + s1b[...]) * dinv
    h2 = (s2a[...] + s2b[...]) * dinv
    h3 = (s3a[...] + s3b[...]) * dinv
    cat = jnp.concatenate([h_ref[...], h1, h2, h3], axis=1)
    out = jnp.dot(cat, w_ref[...], preferred_element_type=jnp.float32,
                  precision=lax.Precision.HIGHEST) + b_ref[...]
    if use_elu:
        out = jnp.where(out > 0, out, jnp.expm1(out))
    out_ref[...] = out
    gnext_ref[...] = out * dinv


def _row_spec(width, half):
    return pl.BlockSpec((ROW_BLK, width), lambda i, h=half: (i + 10 * h, 0))


def _tc_scale_x(degp, x):
    d = x.shape[1]
    return pl.pallas_call(
        _scale_x_body,
        grid=(N_NODES // ROW_BLK,),
        in_specs=[_row_spec(16, 0), _row_spec(16, 1), _row_spec(d, 0)],
        out_specs=_row_spec(d, 0),
        out_shape=jax.ShapeDtypeStruct((N_NODES, d), jnp.float32),
    )(degp, degp, x)


def _tc_rescale(degp, sp):
    d = sp.shape[1]
    return pl.pallas_call(
        _rescale_body,
        grid=(N_NODES // ROW_BLK,),
        in_specs=[_row_spec(16, 0), _row_spec(16, 1),
                  _row_spec(d, 0), _row_spec(d, 1)],
        out_specs=_row_spec(d, 0),
        out_shape=jax.ShapeDtypeStruct((N_NODES, d), jnp.float32),
    )(degp, degp, sp, sp)


def _tc_tag_out(degp, h, s1, s2, s3, w, b, use_elu):
    d = h.shape[1]
    dout = w.shape[1]
    full = lambda *dims: pl.BlockSpec(dims, lambda i, n=len(dims): (0,) * n)
    out, gnext = pl.pallas_call(
        functools.partial(_tag_out_body, use_elu),
        grid=(N_NODES // ROW_BLK,),
        in_specs=[_row_spec(16, 0), _row_spec(16, 1), _row_spec(d, 0),
                  _row_spec(d, 0), _row_spec(d, 1),
                  _row_spec(d, 0), _row_spec(d, 1),
                  _row_spec(d, 0), _row_spec(d, 1),
                  full(4 * d, dout), full(1, dout)],
        out_specs=[_row_spec(dout, 0), _row_spec(dout, 0)],
        out_shape=[jax.ShapeDtypeStruct((N_NODES, dout), jnp.float32),
                   jax.ShapeDtypeStruct((N_NODES, dout), jnp.float32)],
    )(degp, degp, h, s1, s1, s2, s2, s3, s3, w, b.reshape(1, dout))
    return out, gnext


def kernel(x, W1, b1, W2, b2, W3, b3, train_pos_edge_index):
    src = train_pos_edge_index[0].astype(jnp.int32)
    dst = train_pos_edge_index[1].astype(jnp.int32)
    pad = E_PAD - N_EDGES
    srcp = jnp.concatenate([src, jnp.zeros((pad,), jnp.int32)])
    dstp = jnp.concatenate([dst, jnp.full((pad,), N_NODES, jnp.int32)])
    srcg = srcp.reshape(NW * GPW, GROUP)
    dstg = dstp.reshape(NW * GPW, GROUP)
    ones16 = jnp.ones((GROUP, 16), jnp.float32)
    z16 = jnp.zeros((GROUP, 16), jnp.float32)
    z128 = jnp.zeros((GROUP, 128), jnp.float32)
    z64 = jnp.zeros((GROUP, 64), jnp.float32)

    degp = _make_sc_degree()(dstg, ones16, z16)

    prop128 = _make_sc_prop(128)
    prop64 = _make_sc_prop(64)

    def layer(h, g, w, b, zeros, prop, use_elu):
        s1 = prop(g, srcg, dstg, zeros)
        s2 = prop(_tc_rescale(degp, s1), srcg, dstg, zeros)
        s3 = prop(_tc_rescale(degp, s2), srcg, dstg, zeros)
        return _tc_tag_out(degp, h, s1, s2, s3, w, b, use_elu)

    g0 = _tc_scale_x(degp, x)
    h1, g1 = layer(x, g0, W1, b1, z128, prop128, False)
    h2, g2 = layer(h1, g1, W2, b2, z128, prop128, True)
    h3, _ = layer(h2, g2, W3, b3, z64, prop64, False)
    return h3


# speedup vs baseline: 3.2067x; 3.2067x over previous
import functools

import jax
import jax.numpy as jnp
from jax import lax
from jax.experimental import pallas as pl
from jax.experimental.pallas import tpu as pltpu
from jax.experimental.pallas import tpu_sc as plsc

N_NODES = 10000
N_EDGES = 320000
GROUP = 128
NC, NS = 2, 16
NW = NC * NS
GPW = (-(-N_EDGES // (GROUP * NW)) + 7) // 8 * 8
E_PAD = GPW * NW * GROUP
ACC_ROWS = 10240
WB_FULL = N_NODES // GROUP
WB_TAIL = N_NODES - WB_FULL * GROUP
ROW_BLK = 1000


def _sc_mesh():
    return plsc.VectorSubcoreMesh(core_axis_name="c", subcore_axis_name="s",
                                  num_cores=NC, num_subcores=NS)


def _writeback(c, s, acc, stage_v, out_hbm):
    nb = jnp.where(s < WB_FULL % NS, WB_FULL // NS + 1, WB_FULL // NS)

    def wb(t, carry):
        b = t * NS + s
        pltpu.sync_copy(acc.at[pl.ds(b * GROUP, GROUP)], stage_v)
        pltpu.sync_copy(stage_v,
                        out_hbm.at[pl.ds(c * N_NODES + b * GROUP, GROUP)])
        return carry

    lax.fori_loop(0, nb, wb, 0)

    @pl.when(s == NS - 1)
    def _tail():
        base = WB_FULL * GROUP
        pltpu.sync_copy(acc.at[pl.ds(base, WB_TAIL)],
                        stage_v.at[pl.ds(0, WB_TAIL)])
        pltpu.sync_copy(stage_v.at[pl.ds(0, WB_TAIL)],
                        out_hbm.at[pl.ds(c * N_NODES + base, WB_TAIL)])


def _make_sc_prop(d):
    @functools.partial(
        pl.kernel,
        mesh=_sc_mesh(),
        out_type=jax.ShapeDtypeStruct((2 * N_NODES, d), jnp.float32),
        scratch_types=[
            pltpu.VMEM((GPW, GROUP), jnp.int32),
            pltpu.VMEM((GPW, GROUP), jnp.int32),
            pltpu.VMEM((GROUP, d), jnp.float32),
            pltpu.VMEM_SHARED((ACC_ROWS, d), jnp.float32),
            pltpu.SemaphoreType.DMA,
        ],
    )
    def k(g_hbm, src_hbm, dst_hbm, zeros_hbm, out_hbm,
          sidx, didx, rows_v, acc, sem):
        c = lax.axis_index("c")
        s = lax.axis_index("s")
        wid = s * NC + c
        pltpu.sync_copy(zeros_hbm, rows_v)
        for t in range(5):
            pltpu.sync_copy(rows_v, acc.at[pl.ds(s * 640 + t * GROUP, GROUP)])
        plsc.subcore_barrier()
        pltpu.sync_copy(src_hbm.at[pl.ds(wid * GPW, GPW)], sidx)
        pltpu.sync_copy(dst_hbm.at[pl.ds(wid * GPW, GPW)], didx)

        def body(t, carry):
            pltpu.async_copy(g_hbm.at[sidx.at[t]], rows_v, sem).wait()
            pltpu.sync_copy(rows_v, acc.at[didx.at[t]], add=True)
            return carry

        lax.fori_loop(0, GPW, body, 0)
        plsc.subcore_barrier()
        _writeback(c, s, acc, rows_v, out_hbm)

    return k


def _deg_dinv(d0, d1):
    deg = d0[:, 0:1] + d1[:, 0:1]
    return jnp.where(deg > 0, lax.rsqrt(jnp.maximum(deg, 1.0)), 0.0), deg


def _scale_x_body(d0, d1, x_ref, g_ref):
    dinv, _ = _deg_dinv(d0[...], d1[...])
    g_ref[...] = x_ref[...] * dinv


def _rescale_body(d0, d1, s0, s1, g_ref):
    _, deg = _deg_dinv(d0[...], d1[...])
    inv = jnp.where(deg > 0, 1.0 / jnp.maximum(deg, 1.0), 0.0)
    g_ref[...] = (s0[...] + s1[...]) * inv


def _tag_out_body(use_elu, d_use, gnext_pad, d0, d1, h_ref,
                  s1a, s1b, s2a, s2b, s3a, s3b,
                  w_ref, b_ref, out_ref, gnext_ref):
    dinv, _ = _deg_dinv(d0[...], d1[...])
    h1 = (s1a[...] + s1b[...])[:, :d_use] * dinv
    h2 = (s2a[...] + s2b[...])[:, :d_use] * dinv
    h3 = (s3a[...] + s3b[...])[:, :d_use] * dinv
    cat = jnp.concatenate([h_ref[...], h1, h2, h3], axis=1)
    out = jnp.dot(cat, w_ref[...], preferred_element_type=jnp.float32,
                  precision=lax.Precision.HIGHEST) + b_ref[...]
    if use_elu:
        out = jnp.where(out > 0, out, jnp.exp(jnp.minimum(out, 0.0)) - 1.0)
    out_ref[...] = out
    g = out * dinv
    if gnext_pad:
        g = jnp.concatenate(
            [g, jnp.zeros((g.shape[0], gnext_pad), jnp.float32)], axis=1)
    gnext_ref[...] = g


def _row_spec(width, half):
    return pl.BlockSpec((ROW_BLK, width), lambda i, h=half: (i + 10 * h, 0))


def _tc_scale_x(degp, x):
    d = x.shape[1]
    return pl.pallas_call(
        _scale_x_body,
        grid=(N_NODES // ROW_BLK,),
        in_specs=[_row_spec(16, 0), _row_spec(16, 1), _row_spec(d, 0)],
        out_specs=_row_spec(d, 0),
        out_shape=jax.ShapeDtypeStruct((N_NODES, d), jnp.float32),
    )(degp, degp, x)


def _tc_rescale(degp, sp):
    d = sp.shape[1]
    return pl.pallas_call(
        _rescale_body,
        grid=(N_NODES // ROW_BLK,),
        in_specs=[_row_spec(16, 0), _row_spec(16, 1),
                  _row_spec(d, 0), _row_spec(d, 1)],
        out_specs=_row_spec(d, 0),
        out_shape=jax.ShapeDtypeStruct((N_NODES, d), jnp.float32),
    )(degp, degp, sp, sp)


def _tc_tag_out(degp, h, s1, s2, s3, w, b, use_elu, gnext_pad=0):
    d = h.shape[1]
    ds = s1.shape[1]
    dout = w.shape[1]
    gw = dout + gnext_pad
    full = lambda *dims: pl.BlockSpec(dims, lambda i, n=len(dims): (0,) * n)
    out, gnext = pl.pallas_call(
        functools.partial(_tag_out_body, use_elu, d, gnext_pad),
        grid=(N_NODES // ROW_BLK,),
        in_specs=[_row_spec(16, 0), _row_spec(16, 1), _row_spec(d, 0),
                  _row_spec(ds, 0), _row_spec(ds, 1),
                  _row_spec(ds, 0), _row_spec(ds, 1),
                  _row_spec(ds, 0), _row_spec(ds, 1),
                  full(4 * d, dout), full(1, dout)],
        out_specs=[_row_spec(dout, 0), _row_spec(gw, 0)],
        out_shape=[jax.ShapeDtypeStruct((N_NODES, dout), jnp.float32),
                   jax.ShapeDtypeStruct((N_NODES, gw), jnp.float32)],
    )(degp, degp, h, s1, s1, s2, s2, s3, s3, w, b.reshape(1, dout))
    return out, gnext


def kernel(x, W1, b1, W2, b2, W3, b3, train_pos_edge_index):
    src = train_pos_edge_index[0].astype(jnp.int32)
    dst = train_pos_edge_index[1].astype(jnp.int32)
    pad = E_PAD - N_EDGES
    srcp = jnp.concatenate([src, jnp.zeros((pad,), jnp.int32)])
    dstp = jnp.concatenate([dst, jnp.full((pad,), N_NODES, jnp.int32)])
    srcg = srcp.reshape(NW * GPW, GROUP)
    dstg = dstp.reshape(NW * GPW, GROUP)
    z128 = jnp.zeros((GROUP, 128), jnp.float32)

    prop = _make_sc_prop(128)
    degp = prop(jnp.ones((N_NODES, 128), jnp.float32), srcg, dstg, z128)[:, :16]

    def layer(h, g, w, b, use_elu, gnext_pad=0):
        s1 = prop(g, srcg, dstg, z128)
        s2 = prop(_tc_rescale(degp, s1), srcg, dstg, z128)
        s3 = prop(_tc_rescale(degp, s2), srcg, dstg, z128)
        return _tc_tag_out(degp, h, s1, s2, s3, w, b, use_elu, gnext_pad)

    g0 = _tc_scale_x(degp, x)
    h1, g1 = layer(x, g0, W1, b1, False)
    h2, g2 = layer(h1, g1, W2, b2, True, gnext_pad=64)
    h3, _ = layer(h2, g2, W3, b3, False)
    return h3


# speedup vs baseline: 3.4746x; 1.0836x over previous
import functools

import jax
import jax.numpy as jnp
from jax import lax
from jax.experimental import pallas as pl
from jax.experimental.pallas import tpu as pltpu
from jax.experimental.pallas import tpu_sc as plsc

N_NODES = 10000
N_EDGES = 320000
GROUP = 128
NC, NS = 2, 16
NW = NC * NS
GPW = (-(-N_EDGES // (GROUP * NW)) + 7) // 8 * 8
E_PAD = GPW * NW * GROUP
ACC_ROWS = 10240
WB_FULL = N_NODES // GROUP
WB_TAIL = N_NODES - WB_FULL * GROUP
NBUF = 2
IDXC = 40
ROW_BLK = 1000


def _sc_mesh():
    return plsc.VectorSubcoreMesh(core_axis_name="c", subcore_axis_name="s",
                                  num_cores=NC, num_subcores=NS)


def _writeback(c, s, acc, stage_v, out_hbm):
    nb = jnp.where(s < WB_FULL % NS, WB_FULL // NS + 1, WB_FULL // NS)

    def wb(t, carry):
        b = t * NS + s
        pltpu.sync_copy(acc.at[pl.ds(b * GROUP, GROUP)], stage_v)
        pltpu.sync_copy(stage_v,
                        out_hbm.at[pl.ds(c * N_NODES + b * GROUP, GROUP)])
        return carry

    lax.fori_loop(0, nb, wb, 0)

    @pl.when(s == NS - 1)
    def _tail():
        base = WB_FULL * GROUP
        pltpu.sync_copy(acc.at[pl.ds(base, WB_TAIL)],
                        stage_v.at[pl.ds(0, WB_TAIL)])
        pltpu.sync_copy(stage_v.at[pl.ds(0, WB_TAIL)],
                        out_hbm.at[pl.ds(c * N_NODES + base, WB_TAIL)])


def _make_sc_prop(d):
    @functools.partial(
        pl.kernel,
        mesh=_sc_mesh(),
        out_type=jax.ShapeDtypeStruct((2 * N_NODES, d), jnp.float32),
        scratch_types=[
            pltpu.VMEM((IDXC, GROUP), jnp.int32),
            pltpu.VMEM((IDXC, GROUP), jnp.int32),
            pltpu.VMEM((NBUF, GROUP, d), jnp.float32),
            pltpu.VMEM_SHARED((ACC_ROWS, d), jnp.float32),
            pltpu.SemaphoreType.DMA((NBUF,)),
            pltpu.SemaphoreType.DMA((NBUF,)),
        ],
    )
    def k(g_hbm, src_hbm, dst_hbm, zeros_hbm, out_hbm,
          sidx, didx, rows_v, acc, gsem, ssem):
        c = lax.axis_index("c")
        s = lax.axis_index("s")
        wid = s * NC + c
        pltpu.sync_copy(zeros_hbm, rows_v.at[0])
        for t in range(5):
            pltpu.sync_copy(rows_v.at[0],
                            acc.at[pl.ds(s * 640 + t * GROUP, GROUP)])
        plsc.subcore_barrier()

        def g_copy(t, b):
            return pltpu.make_async_copy(
                g_hbm.at[sidx.at[t]], rows_v.at[b], gsem.at[b])

        def s_copy(t, b):
            return pltpu.make_async_copy(
                rows_v.at[b], acc.at[didx.at[t]], ssem.at[b])

        for kc in range(GPW // IDXC):
            base = wid * GPW + kc * IDXC
            pltpu.sync_copy(src_hbm.at[pl.ds(base, IDXC)], sidx)
            pltpu.sync_copy(dst_hbm.at[pl.ds(base, IDXC)], didx)
            for b in range(NBUF):
                g_copy(b, b).start()

            def body(i, carry):
                t0 = i * NBUF
                for b in range(NBUF):
                    g_copy(t0 + b, b).wait()
                    pltpu.async_copy(rows_v.at[b], acc.at[didx.at[t0 + b]],
                                     ssem.at[b], add=True)
                for b in range(NBUF):
                    s_copy(t0 + b, b).wait()

                    @pl.when(t0 + b + NBUF < IDXC)
                    def _refill():
                        g_copy(t0 + b + NBUF, b).start()

                return carry

            lax.fori_loop(0, IDXC // NBUF, body, 0)

        plsc.subcore_barrier()
        _writeback(c, s, acc, rows_v.at[0], out_hbm)

    return k


def _deg_dinv(d0, d1):
    deg = d0[:, 0:1] + d1[:, 0:1]
    return jnp.where(deg > 0, lax.rsqrt(jnp.maximum(deg, 1.0)), 0.0), deg


def _scale_x_body(d0, d1, x_ref, g_ref):
    dinv, _ = _deg_dinv(d0[...], d1[...])
    g_ref[...] = x_ref[...] * dinv


def _rescale_body(d0, d1, s0, s1, g_ref):
    _, deg = _deg_dinv(d0[...], d1[...])
    inv = jnp.where(deg > 0, 1.0 / jnp.maximum(deg, 1.0), 0.0)
    g_ref[...] = (s0[...] + s1[...]) * inv


def _tag_out_body(use_elu, d_use, gnext_pad, d0, d1, h_ref,
                  s1a, s1b, s2a, s2b, s3a, s3b,
                  w_ref, b_ref, out_ref, gnext_ref):
    dinv, _ = _deg_dinv(d0[...], d1[...])
    h1 = (s1a[...] + s1b[...])[:, :d_use] * dinv
    h2 = (s2a[...] + s2b[...])[:, :d_use] * dinv
    h3 = (s3a[...] + s3b[...])[:, :d_use] * dinv
    cat = jnp.concatenate([h_ref[...], h1, h2, h3], axis=1)
    out = jnp.dot(cat, w_ref[...], preferred_element_type=jnp.float32,
                  precision=lax.Precision.HIGHEST) + b_ref[...]
    if use_elu:
        out = jnp.where(out > 0, out, jnp.exp(jnp.minimum(out, 0.0)) - 1.0)
    out_ref[...] = out
    g = out * dinv
    if gnext_pad:
        g = jnp.concatenate(
            [g, jnp.zeros((g.shape[0], gnext_pad), jnp.float32)], axis=1)
    gnext_ref[...] = g


def _row_spec(width, half):
    return pl.BlockSpec((ROW_BLK, width), lambda i, h=half: (i + 10 * h, 0))


def _tc_scale_x(degp, x):
    d = x.shape[1]
    return pl.pallas_call(
        _scale_x_body,
        grid=(N_NODES // ROW_BLK,),
        in_specs=[_row_spec(16, 0), _row_spec(16, 1), _row_spec(d, 0)],
        out_specs=_row_spec(d, 0),
        out_shape=jax.ShapeDtypeStruct((N_NODES, d), jnp.float32),
    )(degp, degp, x)


def _tc_rescale(degp, sp):
    d = sp.shape[1]
    return pl.pallas_call(
        _rescale_body,
        grid=(N_NODES // ROW_BLK,),
        in_specs=[_row_spec(16, 0), _row_spec(16, 1),
                  _row_spec(d, 0), _row_spec(d, 1)],
        out_specs=_row_spec(d, 0),
        out_shape=jax.ShapeDtypeStruct((N_NODES, d), jnp.float32),
    )(degp, degp, sp, sp)


def _tc_tag_out(degp, h, s1, s2, s3, w, b, use_elu, gnext_pad=0):
    d = h.shape[1]
    ds = s1.shape[1]
    dout = w.shape[1]
    gw = dout + gnext_pad
    full = lambda *dims: pl.BlockSpec(dims, lambda i, n=len(dims): (0,) * n)
    out, gnext = pl.pallas_call(
        functools.partial(_tag_out_body, use_elu, d, gnext_pad),
        grid=(N_NODES // ROW_BLK,),
        in_specs=[_row_spec(16, 0), _row_spec(16, 1), _row_spec(d, 0),
                  _row_spec(ds, 0), _row_spec(ds, 1),
                  _row_spec(ds, 0), _row_spec(ds, 1),
                  _row_spec(ds, 0), _row_spec(ds, 1),
                  full(4 * d, dout), full(1, dout)],
        out_specs=[_row_spec(dout, 0), _row_spec(gw, 0)],
        out_shape=[jax.ShapeDtypeStruct((N_NODES, dout), jnp.float32),
                   jax.ShapeDtypeStruct((N_NODES, gw), jnp.float32)],
    )(degp, degp, h, s1, s1, s2, s2, s3, s3, w, b.reshape(1, dout))
    return out, gnext


def kernel(x, W1, b1, W2, b2, W3, b3, train_pos_edge_index):
    src = train_pos_edge_index[0].astype(jnp.int32)
    dst = train_pos_edge_index[1].astype(jnp.int32)
    pad = E_PAD - N_EDGES
    srcp = jnp.concatenate([src, jnp.zeros((pad,), jnp.int32)])
    dstp = jnp.concatenate([dst, jnp.full((pad,), N_NODES, jnp.int32)])
    srcg = srcp.reshape(NW * GPW, GROUP)
    dstg = dstp.reshape(NW * GPW, GROUP)
    z128 = jnp.zeros((GROUP, 128), jnp.float32)

    prop = _make_sc_prop(128)
    degp = prop(jnp.ones((N_NODES, 128), jnp.float32), srcg, dstg, z128)[:, :16]

    def layer(h, g, w, b, use_elu, gnext_pad=0):
        s1 = prop(g, srcg, dstg, z128)
        s2 = prop(_tc_rescale(degp, s1), srcg, dstg, z128)
        s3 = prop(_tc_rescale(degp, s2), srcg, dstg, z128)
        return _tc_tag_out(degp, h, s1, s2, s3, w, b, use_elu, gnext_pad)

    g0 = _tc_scale_x(degp, x)
    h1, g1 = layer(x, g0, W1, b1, False)
    h2, g2 = layer(h1, g1, W2, b2, True, gnext_pad=64)
    h3, _ = layer(h2, g2, W3, b3, False)
    return h3


# speedup vs baseline: 5.2699x; 1.5167x over previous
import functools

import jax
import jax.numpy as jnp
from jax import lax
from jax.experimental import pallas as pl
from jax.experimental.pallas import tpu as pltpu
from jax.experimental.pallas import tpu_sc as plsc

N_NODES = 10000
N_EDGES = 320000
GROUP = 128
DH = 64
NC, NS = 2, 16
NW = NC * NS
GPW = (-(-N_EDGES // (GROUP * NW)) + 7) // 8 * 8
E_PAD = GPW * NW * GROUP
NWG = NW * GPW
GPS = NWG // NS
ACC_ROWS = 10240
WB_FULL = N_NODES // GROUP
WB_TAIL = N_NODES - WB_FULL * GROUP
NBUF = 4
IDXC = 40
INV_ROWS = ACC_ROWS // 16
ROW_BLK = 1000


def _sc_mesh():
    return plsc.VectorSubcoreMesh(core_axis_name="c", subcore_axis_name="s",
                                  num_cores=NC, num_subcores=NS)


def _scale_block(stage_v, inv_v, nrows):
    for j16 in range(nrows // 16):
        q = inv_v[j16]
        for j in range(16):
            spl = q.at[jnp.full((16,), j, jnp.int32)].get(
                mode="promise_in_bounds")
            r = j16 * 16 + j
            for cc in range(DH // 16):
                sl = pl.ds(cc * 16, 16)
                stage_v[r, sl] = stage_v[r, sl] * spl


def _writeback(c, s, acc, stage_v, inv_v, inv_hbm, out_hbm, scale):
    nb = jnp.where(s < WB_FULL % NS, WB_FULL // NS + 1, WB_FULL // NS)

    def wb(t, carry):
        b = t * NS + s
        pltpu.sync_copy(acc.at[pl.ds(b * GROUP, GROUP)], stage_v)
        if scale:
            pltpu.sync_copy(inv_hbm.at[pl.ds(b * 8, 8)], inv_v)
            _scale_block(stage_v, inv_v, GROUP)
        pltpu.sync_copy(stage_v,
                        out_hbm.at[pl.ds(c * N_NODES + b * GROUP, GROUP)])
        return carry

    lax.fori_loop(0, nb, wb, 0)

    @pl.when(s == NS - 1)
    def _tail():
        base = WB_FULL * GROUP
        pltpu.sync_copy(acc.at[pl.ds(base, WB_TAIL)],
                        stage_v.at[pl.ds(0, WB_TAIL)])
        if scale:
            pltpu.sync_copy(inv_hbm.at[pl.ds(WB_FULL * 8, 8)], inv_v)
            _scale_block(stage_v, inv_v, WB_TAIL)
        pltpu.sync_copy(stage_v.at[pl.ds(0, WB_TAIL)],
                        out_hbm.at[pl.ds(c * N_NODES + base, WB_TAIL)])


def _make_sc_layer(nhops):
    outs = [jax.ShapeDtypeStruct((2 * N_NODES, DH), jnp.float32)
            for _ in range(nhops)]

    @functools.partial(
        pl.kernel,
        mesh=_sc_mesh(),
        out_type=outs if nhops > 1 else outs[0],
        scratch_types=[
            pltpu.VMEM((IDXC, GROUP), jnp.int32),
            pltpu.VMEM((IDXC, GROUP), jnp.int32),
            pltpu.VMEM((NBUF, GROUP, DH), jnp.float32),
            pltpu.VMEM((GROUP, DH), jnp.float32),
            pltpu.VMEM((8, 16), jnp.float32),
            pltpu.VMEM_SHARED((ACC_ROWS, DH), jnp.float32),
            pltpu.SemaphoreType.DMA((NBUF,)),
            pltpu.SemaphoreType.DMA((NBUF,)),
        ],
        compiler_params=pltpu.CompilerParams(use_tc_tiling_on_sc=False),
    )
    def k(t0_hbm, src_hbm, dst_hbm, zeros_hbm, inv_hbm, *out_and_scratch):
        out_hbms = out_and_scratch[:nhops]
        (sidx, didx, rows_v, stage_v, inv_v, acc,
         gsem, ssem) = out_and_scratch[nhops:]
        c = lax.axis_index("c")
        s = lax.axis_index("s")

        def g_copy(table, t, b):
            return pltpu.make_async_copy(
                table.at[sidx.at[t]], rows_v.at[b], gsem.at[b])

        def s_copy(t, b):
            return pltpu.make_async_copy(
                rows_v.at[b], acc.at[didx.at[t]], ssem.at[b])

        for h in range(nhops):
            table = t0_hbm if h == 0 else out_hbms[h - 1]
            pltpu.sync_copy(zeros_hbm, stage_v)
            for t in range(5):
                pltpu.sync_copy(stage_v,
                                acc.at[pl.ds(s * 640 + t * GROUP, GROUP)])
            plsc.subcore_barrier()

            for kc in range(GPS // IDXC):
                base = s * GPS + kc * IDXC
                pltpu.sync_copy(src_hbm.at[pl.ds(c * NWG + base, IDXC)], sidx)
                pltpu.sync_copy(dst_hbm.at[pl.ds(base, IDXC)], didx)
                for b in range(NBUF):
                    g_copy(table, b, b).start()

                def body(i, carry):
                    t0 = i * NBUF
                    for b in range(NBUF):
                        g_copy(table, t0 + b, b).wait()
                        pltpu.async_copy(rows_v.at[b],
                                         acc.at[didx.at[t0 + b]],
                                         ssem.at[b], add=True)
                    for b in range(NBUF):
                        s_copy(t0 + b, b).wait()

                        @pl.when(t0 + b + NBUF < IDXC)
                        def _refill():
                            g_copy(table, t0 + b + NBUF, b).start()
                    return carry

                lax.fori_loop(0, IDXC // NBUF, body, 0)

            plsc.subcore_barrier()
            _writeback(c, s, acc, stage_v, inv_v, inv_hbm, out_hbms[h],
                       scale=(h < nhops - 1))
            plsc.subcore_barrier()

    return k


def _deg_prep_body(deg_ref, x_ref, g0_ref, inv2_ref, sq_ref):
    deg = deg_ref[:, 0:1]
    pos = deg > 0
    dinv = jnp.where(pos, lax.rsqrt(jnp.maximum(deg, 1.0)), 0.0)
    g0 = x_ref[...] * dinv
    g0_ref[...] = jnp.concatenate([g0[:, :DH], g0[:, DH:]], axis=0)
    inv = jnp.where(pos, 1.0 / jnp.maximum(deg, 1.0), 0.0)
    inv2_ref[...] = jnp.broadcast_to(inv, (N_NODES, 16))
    sq = jnp.where(pos, jnp.sqrt(jnp.maximum(deg, 1.0)), 0.0)
    sq_ref[...] = jnp.broadcast_to(sq, (N_NODES, 16))


def _tc_deg_prep(deg64, x):
    full = lambda *dims: pl.BlockSpec(dims, lambda n=len(dims): (0,) * n)
    return pl.pallas_call(
        _deg_prep_body,
        grid=(),
        in_specs=[full(N_NODES, DH), full(N_NODES, 128)],
        out_specs=[full(2 * N_NODES, DH), full(N_NODES, 16),
                   full(N_NODES, 16)],
        out_shape=[jax.ShapeDtypeStruct((2 * N_NODES, DH), jnp.float32),
                   jax.ShapeDtypeStruct((N_NODES, 16), jnp.float32),
                   jax.ShapeDtypeStruct((N_NODES, 16), jnp.float32)],
    )(deg64, x)


def _tag_out_body(use_elu, emit_g, d_in, d0, sq_ref, h_ref,
                  t1a, t1b, t2a, t2b, s3a, s3b, w_ref, b_ref,
                  out_ref, g_ref):
    deg = d0[:, 0:1]
    pos = deg > 0
    dinv = jnp.where(pos, lax.rsqrt(jnp.maximum(deg, 1.0)), 0.0)
    sq = sq_ref[:, 0:1]
    if d_in == 128:
        h1 = jnp.concatenate([t1a[...], t1b[...]], axis=1) * sq
        h2 = jnp.concatenate([t2a[...], t2b[...]], axis=1) * sq
        h3 = jnp.concatenate([s3a[...], s3b[...]], axis=1) * dinv
    else:
        h1 = t1a[...] * sq
        h2 = t2a[...] * sq
        h3 = s3a[...] * dinv
    cat = jnp.concatenate([h_ref[...], h1, h2, h3], axis=1)
    out = jnp.dot(cat, w_ref[...], preferred_element_type=jnp.float32,
                  precision=lax.Precision.HIGHEST) + b_ref[...]
    if use_elu:
        out = jnp.where(out > 0, out, jnp.exp(jnp.minimum(out, 0.0)) - 1.0)
    out_ref[...] = out
    if emit_g:
        g_ref[...] = out * dinv


def _tc_tag_out(deg64, sqdeg, h, t1, t2, s3, w, b, use_elu, emit_g):
    d = h.shape[1]
    dout = w.shape[1]
    full = lambda *dims: pl.BlockSpec(dims, lambda i, n=len(dims): (0,) * n)
    halves = [_row_spec(DH, 0), _row_spec(DH, 1)] if d == 128 else \
             [_row_spec(DH, 0), _row_spec(DH, 0)]
    specs = [_row_spec(DH, 0), _row_spec(16, 0), _row_spec(d, 0)]
    for arr in range(3):
        specs += halves
    specs += [full(4 * d, dout), full(1, dout)]
    out_shapes = [jax.ShapeDtypeStruct((N_NODES, dout), jnp.float32),
                  jax.ShapeDtypeStruct((N_NODES, dout), jnp.float32)]
    out, g = pl.pallas_call(
        functools.partial(_tag_out_body, use_elu, emit_g, d),
        grid=(N_NODES // ROW_BLK,),
        in_specs=specs,
        out_specs=[_row_spec(dout, 0), _row_spec(dout, 0)],
        out_shape=out_shapes,
    )(deg64, sqdeg, h, t1, t1, t2, t2, s3, s3, w, b.reshape(1, dout))
    return out, g


def _row_spec(width, half):
    return pl.BlockSpec((ROW_BLK, width),
                        lambda i, h=half: (i + (N_NODES // ROW_BLK) * h, 0))


def kernel(x, W1, b1, W2, b2, W3, b3, train_pos_edge_index):
    src = train_pos_edge_index[0].astype(jnp.int32)
    dst = train_pos_edge_index[1].astype(jnp.int32)
    pad = E_PAD - N_EDGES
    srcp = jnp.concatenate([src, jnp.zeros((pad,), jnp.int32)])
    dstp = jnp.concatenate([dst, jnp.full((pad,), N_NODES, jnp.int32)])
    srcg = srcp.reshape(NWG, GROUP)
    srcO = jnp.concatenate([srcg, srcg + N_NODES], axis=0)
    dstg = dstp.reshape(NWG, GROUP)
    z64 = jnp.zeros((GROUP, DH), jnp.float32)
    inv_dummy = jnp.zeros((INV_ROWS, 16), jnp.float32)

    deg64 = _make_sc_layer(1)(
        jnp.ones((2 * N_NODES, DH), jnp.float32), srcO, dstg, z64, inv_dummy)
    deg64 = deg64[:N_NODES]
    g0, invb, sqdeg = _tc_deg_prep(deg64, x)
    inv2 = jnp.concatenate(
        [invb[:, 0], jnp.zeros((ACC_ROWS - N_NODES,), jnp.float32)]
    ).reshape(INV_ROWS, 16)

    layer3 = _make_sc_layer(3)

    t1, t2, s3 = layer3(g0, srcO, dstg, z64, inv2)
    h1, gf1 = _tc_tag_out(deg64, sqdeg, x, t1, t2, s3, W1, b1,
                          use_elu=False, emit_g=True)
    g1 = jnp.concatenate([gf1[:, :DH], gf1[:, DH:]], axis=0)

    t1, t2, s3 = layer3(g1, srcO, dstg, z64, inv2)
    h2, gf2 = _tc_tag_out(deg64, sqdeg, h1, t1, t2, s3, W2, b2,
                          use_elu=True, emit_g=True)
    g2 = jnp.concatenate([gf2, jnp.zeros((N_NODES, DH), jnp.float32)], axis=0)

    t1, t2, s3 = layer3(g2, srcO, dstg, z64, inv2)
    h3, _ = _tc_tag_out(deg64, sqdeg, h2, t1, t2, s3, W3, b3,
                        use_elu=False, emit_g=False)
    return h3
